# baseline (speedup 1.0000x reference)
import jax
import jax.numpy as jnp
from jax import lax
from jax.experimental import pallas as pl
from jax.experimental.pallas import tpu as pltpu
from jax.experimental.pallas import tpu_sc as plsc

VOCAB = 100000
EMBED = 64
NTOK = 200
BATCH = 4096

NC = 2
NS = 16
NW = NC * NS
TT = NTOK // 8
TI = 8
BL = 128
EH = EMBED // 8
LANES = 16
NBUF = 4


def _body(tokens_hbm, table_hbm, pos_hbm, out_hbm, idx_v, rows_v, tr_v,
          pos_v, g0, g1, g2, g3, w0, w1, w2, w3):
    sem_g = (g0, g1, g2, g3)
    sem_w = (w0, w1, w2, w3)
    wid = lax.axis_index("s") * NC + lax.axis_index("c")
    pltpu.sync_copy(tokens_hbm.at[:, wid], idx_v)
    pltpu.sync_copy(pos_hbm, pos_v)

    lane = lax.iota(jnp.int32, LANES)
    eh_c = [(c * LANES + lane) >> 3 for c in range(EMBED // LANES)]
    el_c = [(c * LANES + lane) & 7 for c in range(EMBED // LANES)]

    def start_gather(t, b):
        pltpu.async_copy(table_hbm.at[idx_v.at[t // TI, t % TI]],
                         rows_v.at[b], sem_g[b])

    def wait_gather(t, b):
        pltpu.make_async_copy(table_hbm.at[idx_v.at[t // TI, t % TI]],
                              rows_v.at[b], sem_g[b]).wait()

    def start_write(t, b):
        pltpu.async_copy(tr_v.at[b], out_hbm.at[t, :, wid], sem_w[b])

    def wait_write(t, b):
        pltpu.make_async_copy(tr_v.at[b], out_hbm.at[t, :, wid],
                              sem_w[b]).wait()

    def transpose_add(t, b):
        pvecs = tuple(pos_v[t, pl.ds(c * LANES, LANES)]
                      for c in range(EMBED // LANES))

        def rbody(r, carry):
            blv = jnp.full((LANES,), r, jnp.int32)
            for c in range(EMBED // LANES):
                v = rows_v[b, r, pl.ds(c * LANES, LANES)] + carry[c]
                plsc.store_scatter(tr_v.at[b], [eh_c[c], el_c[c], blv], v)
            return carry

        lax.fori_loop(0, BL, rbody, pvecs, unroll=4)

    def body(t, b, prefetch, reclaim):
        if prefetch:
            start_gather(t + NBUF - 1, (b + NBUF - 1) % NBUF)
        wait_gather(t, b)
        if reclaim:
            wait_write(t - NBUF, b)
        transpose_add(t, b)
        start_write(t, b)

    for t in range(NBUF - 1):
        start_gather(t, t)
    for t in range(NBUF):
        body(t, t, prefetch=True, reclaim=False)

    def outer(k, carry):
        t0 = NBUF + NBUF * k
        for j in range(NBUF):
            body(t0 + j, j, prefetch=True, reclaim=True)
        return carry

    lax.fori_loop(0, 48, outer, 0)
    body(196, 0, prefetch=True, reclaim=True)
    body(197, 1, prefetch=False, reclaim=True)
    body(198, 2, prefetch=False, reclaim=True)
    body(199, 3, prefetch=False, reclaim=True)
    for t in range(NTOK - NBUF, NTOK):
        wait_write(t, t % NBUF)


def kernel(tokens, token_embedding, positional_embedding):
    tok_p = (tokens.astype(jnp.int32).T
             .reshape(TT, TI, NW, BL).transpose(0, 2, 1, 3))
    grid_kernel = pl.kernel(
        _body,
        out_type=jax.ShapeDtypeStruct((NTOK, EH, NW, 8, BL), jnp.float32),
        mesh=plsc.VectorSubcoreMesh(core_axis_name="c", subcore_axis_name="s"),
        compiler_params=pltpu.CompilerParams(use_tc_tiling_on_sc=False,
                                             needs_layout_passes=False),
        scratch_types=[
            pltpu.VMEM((TT, TI, BL), jnp.int32),
            pltpu.VMEM((NBUF, BL, EMBED), jnp.float32),
            pltpu.VMEM((NBUF, EH, 8, BL), jnp.float32),
            pltpu.VMEM((NTOK, EMBED), jnp.float32),
        ] + [pltpu.SemaphoreType.DMA] * (2 * NBUF),
    )
    out5 = grid_kernel(tok_p, token_embedding, positional_embedding)
    return out5.transpose(2, 4, 0, 1, 3).reshape(BATCH, NTOK, EMBED)


# speedup vs baseline: 5.3318x; 5.3318x over previous
import jax
import jax.numpy as jnp
from jax import lax
from jax.experimental import pallas as pl
from jax.experimental.pallas import tpu as pltpu
from jax.experimental.pallas import tpu_sc as plsc

VOCAB = 100000
EMBED = 64
NTOK = 200
BATCH = 4096

NC = 2
NS = 16
NW = NC * NS
TT = NTOK // 8
TI = 8
BL = 128
EH = EMBED // 8
LANES = 16
NBUF = 4


def _body(tokens_hbm, table_hbm, pos_hbm, out_hbm, idx_v, rows_v, tr_v,
          pos_v, g0, g1, g2, g3, w0, w1, w2, w3):
    sem_g = (g0, g1, g2, g3)
    sem_w = (w0, w1, w2, w3)
    wid = lax.axis_index("s") * NC + lax.axis_index("c")
    pltpu.sync_copy(tokens_hbm.at[:, wid], idx_v)
    pltpu.sync_copy(pos_hbm, pos_v)

    lane = lax.iota(jnp.int32, LANES)
    eh_c = [(c * LANES + lane) >> 3 for c in range(EMBED // LANES)]
    el_c = [(c * LANES + lane) & 7 for c in range(EMBED // LANES)]

    def start_gather(t, b):
        pltpu.async_copy(table_hbm.at[idx_v.at[t // TI, t % TI]],
                         rows_v.at[b], sem_g[b])

    def wait_gather(t, b):
        pltpu.make_async_copy(table_hbm.at[idx_v.at[t // TI, t % TI]],
                              rows_v.at[b], sem_g[b]).wait()

    def start_write(t, b):
        pltpu.async_copy(tr_v.at[b, :, :, pl.ds(0, BL)],
                         out_hbm.at[t, :, wid], sem_w[b])

    def wait_write(t, b):
        pltpu.make_async_copy(tr_v.at[b, :, :, pl.ds(0, BL)],
                              out_hbm.at[t, :, wid], sem_w[b]).wait()

    def transpose_add(t, b):
        pvecs = tuple(pos_v[t, pl.ds(c * LANES, LANES)]
                      for c in range(EMBED // LANES))

        @plsc.parallel_loop(0, BL, unroll=4, carry=pvecs)
        def rbody(r, carry):
            blv = jnp.full((LANES,), r, jnp.int32)
            for c in range(EMBED // LANES):
                v = rows_v[b, r, pl.ds(c * LANES, LANES)] + carry[c]
                plsc.store_scatter(tr_v.at[b], [eh_c[c], el_c[c], blv], v)
            return carry

    def body(t, b, prefetch, reclaim):
        if prefetch:
            start_gather(t + NBUF - 1, (b + NBUF - 1) % NBUF)
        wait_gather(t, b)
        if reclaim:
            wait_write(t - NBUF, b)
        transpose_add(t, b)
        start_write(t, b)

    for t in range(NBUF - 1):
        start_gather(t, t)
    for t in range(NBUF):
        body(t, t, prefetch=True, reclaim=False)

    def outer(k, carry):
        t0 = NBUF + NBUF * k
        for j in range(NBUF):
            body(t0 + j, j, prefetch=True, reclaim=True)
        return carry

    lax.fori_loop(0, 48, outer, 0)
    body(196, 0, prefetch=True, reclaim=True)
    body(197, 1, prefetch=False, reclaim=True)
    body(198, 2, prefetch=False, reclaim=True)
    body(199, 3, prefetch=False, reclaim=True)
    for t in range(NTOK - NBUF, NTOK):
        wait_write(t, t % NBUF)


def kernel(tokens, token_embedding, positional_embedding):
    tok_p = (tokens.astype(jnp.int32).T
             .reshape(TT, TI, NW, BL).transpose(0, 2, 1, 3))
    grid_kernel = pl.kernel(
        _body,
        out_type=jax.ShapeDtypeStruct((NTOK, EH, NW, 8, BL), jnp.float32),
        mesh=plsc.VectorSubcoreMesh(core_axis_name="c", subcore_axis_name="s"),
        compiler_params=pltpu.CompilerParams(use_tc_tiling_on_sc=False,
                                             needs_layout_passes=False),
        scratch_types=[
            pltpu.VMEM((TT, TI, BL), jnp.int32),
            pltpu.VMEM((NBUF, BL, EMBED), jnp.float32),
            pltpu.VMEM((NBUF, EH, 8, BL + 1), jnp.float32),
            pltpu.VMEM((NTOK, EMBED), jnp.float32),
        ] + [pltpu.SemaphoreType.DMA] * (2 * NBUF),
    )
    out5 = grid_kernel(tok_p, token_embedding, positional_embedding)
    return out5.transpose(2, 4, 0, 1, 3).reshape(BATCH, NTOK, EMBED)


# speedup vs baseline: 5.3368x; 1.0010x over previous
import jax
import jax.numpy as jnp
from jax import lax
from jax.experimental import pallas as pl
from jax.experimental.pallas import tpu as pltpu
from jax.experimental.pallas import tpu_sc as plsc

VOCAB = 100000
EMBED = 64
NTOK = 200
BATCH = 4096

NC = 2
NS = 16
NW = NC * NS
TT = NTOK // 8
TI = 8
BL = 128
EH = EMBED // 8
LANES = 16
NBUF = 4


def _body(tokens_hbm, table_hbm, pos_hbm, out_hbm, idx_v, rows_v, tr_v,
          pos_v, g0, g1, g2, g3, w0, w1, w2, w3):
    sem_g = (g0, g1, g2, g3)
    sem_w = (w0, w1, w2, w3)
    wid = lax.axis_index("s") * NC + lax.axis_index("c")
    pltpu.sync_copy(tokens_hbm.at[:, wid], idx_v)
    pltpu.sync_copy(pos_hbm, pos_v)

    lane = lax.iota(jnp.int32, LANES)
    eh_c = [(c * LANES + lane) >> 3 for c in range(EMBED // LANES)]
    el_c = [(c * LANES + lane) & 7 for c in range(EMBED // LANES)]

    def start_gather(t, b):
        pltpu.async_copy(table_hbm.at[idx_v.at[t // TI, t % TI]],
                         rows_v.at[b], sem_g[b])

    def wait_gather(t, b):
        pltpu.make_async_copy(table_hbm.at[idx_v.at[t // TI, t % TI]],
                              rows_v.at[b], sem_g[b]).wait()

    def start_write(t, b):
        pltpu.async_copy(tr_v.at[b, :, :, pl.ds(0, BL)],
                         out_hbm.at[t, :, wid], sem_w[b])

    def wait_write(t, b):
        pltpu.make_async_copy(tr_v.at[b, :, :, pl.ds(0, BL)],
                              out_hbm.at[t, :, wid], sem_w[b]).wait()

    def transpose_add(t, b):
        pvecs = tuple(pos_v[t, pl.ds(c * LANES, LANES)]
                      for c in range(EMBED // LANES))

        @plsc.parallel_loop(0, BL, unroll=2, carry=pvecs)
        def rbody(r, carry):
            blv = jnp.full((LANES,), r, jnp.int32)
            for c in range(EMBED // LANES):
                v = rows_v[b, r, pl.ds(c * LANES, LANES)] + carry[c]
                plsc.store_scatter(tr_v.at[b], [eh_c[c], el_c[c], blv], v)
            return carry

    def body(t, b, prefetch, reclaim):
        if prefetch:
            start_gather(t + NBUF - 1, (b + NBUF - 1) % NBUF)
        wait_gather(t, b)
        if reclaim:
            wait_write(t - NBUF, b)
        transpose_add(t, b)
        start_write(t, b)

    for t in range(NBUF - 1):
        start_gather(t, t)
    for t in range(NBUF):
        body(t, t, prefetch=True, reclaim=False)

    def outer(k, carry):
        t0 = NBUF + NBUF * k
        for j in range(NBUF):
            body(t0 + j, j, prefetch=True, reclaim=True)
        return carry

    lax.fori_loop(0, 48, outer, 0)
    body(196, 0, prefetch=True, reclaim=True)
    body(197, 1, prefetch=False, reclaim=True)
    body(198, 2, prefetch=False, reclaim=True)
    body(199, 3, prefetch=False, reclaim=True)
    for t in range(NTOK - NBUF, NTOK):
        wait_write(t, t % NBUF)


def kernel(tokens, token_embedding, positional_embedding):
    tok_p = (tokens.astype(jnp.int32).T
             .reshape(TT, TI, NW, BL).transpose(0, 2, 1, 3))
    grid_kernel = pl.kernel(
        _body,
        out_type=jax.ShapeDtypeStruct((NTOK, EH, NW, 8, BL), jnp.float32),
        mesh=plsc.VectorSubcoreMesh(core_axis_name="c", subcore_axis_name="s"),
        compiler_params=pltpu.CompilerParams(use_tc_tiling_on_sc=False,
                                             needs_layout_passes=False),
        scratch_types=[
            pltpu.VMEM((TT, TI, BL), jnp.int32),
            pltpu.VMEM((NBUF, BL, EMBED), jnp.float32),
            pltpu.VMEM((NBUF, EH, 8, BL + 1), jnp.float32),
            pltpu.VMEM((NTOK, EMBED), jnp.float32),
        ] + [pltpu.SemaphoreType.DMA] * (2 * NBUF),
    )
    out5 = grid_kernel(tok_p, token_embedding, positional_embedding)
    return out5.transpose(2, 4, 0, 1, 3).reshape(BATCH, NTOK, EMBED)
